# Initial kernel scaffold; baseline (speedup 1.0000x reference)
import jax
import jax.numpy as jnp
from jax.experimental import pallas as pl


def kernel(x, edge_index, W0, b0, W1, b1, W2, b2):
    raise NotImplementedError("write your pallas kernel here")


# speedup vs baseline: 20.8254x; 20.8254x over previous
import functools

import jax
import jax.numpy as jnp
from jax import lax
from jax.experimental import pallas as pl
from jax.experimental.pallas import tpu as pltpu
from jax.experimental.pallas import tpu_sc as plsc

N = 10000
D = 128
E = 320000
NC = 2
NS = 16
NW = NC * NS
EPW = E // NW
CHUNK = 80
NCH = EPW // CHUNK
NPAD = 10240
RPS = NPAD // NS
HP = 640
HTOT = NS * HP
BR = 2000


def _mesh():
    return plsc.VectorSubcoreMesh(
        core_axis_name="c", subcore_axis_name="s",
        num_cores=NC, num_subcores=NS)


@functools.lru_cache(maxsize=None)
def _deg_kernel():
    def body(idx_hbm, out_hbm, *, idx_all, ones_v, z_v, hist):
        zero16 = jnp.broadcast_to(jnp.float32(0.0), (16,))
        ones16 = jnp.broadcast_to(jnp.float32(1.0), (16,))
        c = lax.axis_index("c")
        s = lax.axis_index("s")
        w = c * NS + s
        pltpu.sync_copy(idx_hbm.at[w], idx_all)
        for j in range(CHUNK // 16):
            ones_v[pl.ds(j * 16, 16)] = ones16

        def zfill(i, carry):
            z_v[pl.ds(i * 16, 16)] = zero16
            return carry
        lax.fori_loop(0, HP // 16, zfill, 0)
        pltpu.sync_copy(z_v, hist.at[pl.ds(s * HP, HP)])
        plsc.subcore_barrier()

        def step(j, carry):
            pltpu.sync_copy(ones_v, hist.at[idx_all.at[j, 1]], add=True)
            return carry
        lax.fori_loop(0, NCH, step, 0)
        plsc.subcore_barrier()
        pltpu.sync_copy(hist.at[pl.ds(s * HP, HP)], out_hbm.at[pl.ds(w * HP, HP)])

    return pl.kernel(
        body,
        out_type=jax.ShapeDtypeStruct((NC * HTOT,), jnp.float32),
        mesh=_mesh(),
        scratch_types=dict(
            idx_all=pltpu.VMEM((NCH, 2, CHUNK), jnp.int32),
            ones_v=pltpu.VMEM((CHUNK,), jnp.float32),
            z_v=pltpu.VMEM((HP,), jnp.float32),
            hist=pltpu.VMEM_SHARED((HTOT,), jnp.float32),
        ),
    )


@functools.lru_cache(maxsize=None)
def _agg_kernel():
    nfull = RPS // CHUNK

    def body(h_hbm, idx_hbm, out_hbm, *,
             ib_a, ib_b, buf_a, buf_b, acc, sem_i, sem_a, sem_b):
        zero16 = jnp.broadcast_to(jnp.float32(0.0), (16,))
        c = lax.axis_index("c")
        s = lax.axis_index("s")
        w = c * NS + s

        def zrow(i, carry):
            for j in range(D // 16):
                buf_a[i, pl.ds(j * 16, 16)] = zero16
            return carry
        lax.fori_loop(0, CHUNK, zrow, 0)
        base = s * RPS
        for k in range(nfull):
            pltpu.sync_copy(buf_a, acc.at[pl.ds(base + k * CHUNK, CHUNK)])

        pltpu.sync_copy(idx_hbm.at[w, 0], ib_a)
        pltpu.async_copy(h_hbm.at[ib_a.at[0]], buf_a, sem_a)
        pltpu.async_copy(idx_hbm.at[w, 1], ib_b, sem_i)
        plsc.subcore_barrier()

        def step(i, carry):
            j0 = 2 * i
            pltpu.make_async_copy(idx_hbm.at[w, 0], ib_b, sem_i).wait()
            pltpu.make_async_copy(h_hbm.at[ib_a.at[0]], buf_a, sem_a).wait()
            pltpu.async_copy(h_hbm.at[ib_b.at[0]], buf_b, sem_b)
            pltpu.sync_copy(buf_a, acc.at[ib_a.at[1]], add=True)
            pltpu.async_copy(idx_hbm.at[w, j0 + 2], ib_a, sem_i)

            pltpu.make_async_copy(idx_hbm.at[w, 0], ib_a, sem_i).wait()
            pltpu.make_async_copy(h_hbm.at[ib_b.at[0]], buf_b, sem_b).wait()
            pltpu.async_copy(h_hbm.at[ib_a.at[0]], buf_a, sem_a)
            pltpu.sync_copy(buf_b, acc.at[ib_b.at[1]], add=True)
            jn = lax.min(j0 + 3, NCH - 1)
            pltpu.async_copy(idx_hbm.at[w, jn], ib_b, sem_i)
            return carry
        lax.fori_loop(0, NCH // 2, step, 0)
        pltpu.make_async_copy(idx_hbm.at[w, 0], ib_b, sem_i).wait()
        pltpu.make_async_copy(h_hbm.at[ib_a.at[0]], buf_a, sem_a).wait()
        pltpu.sync_copy(buf_a, acc.at[ib_a.at[1]], add=True)
        plsc.subcore_barrier()

        for k in range(nfull):
            off = base + k * CHUNK
            pltpu.sync_copy(acc.at[pl.ds(off, CHUNK)],
                            out_hbm.at[c, pl.ds(off, CHUNK)])

    return pl.kernel(
        body,
        out_type=jax.ShapeDtypeStruct((NC, NPAD, D), jnp.float32),
        mesh=_mesh(),
        scratch_types=dict(
            ib_a=pltpu.VMEM((2, CHUNK), jnp.int32),
            ib_b=pltpu.VMEM((2, CHUNK), jnp.int32),
            buf_a=pltpu.VMEM((CHUNK, D), jnp.float32),
            buf_b=pltpu.VMEM((CHUNK, D), jnp.float32),
            acc=pltpu.VMEM_SHARED((NPAD, D), jnp.float32),
            sem_i=pltpu.SemaphoreType.DMA,
            sem_a=pltpu.SemaphoreType.DMA,
            sem_b=pltpu.SemaphoreType.DMA,
        ),
    )


def _dinv(hist):
    def body(hist_ref, o_ref):
        h = hist_ref[pl.ds(0, HTOT)] + hist_ref[pl.ds(HTOT, HTOT)]
        o_ref[...] = lax.rsqrt(1.0 + h)
    return pl.pallas_call(
        body,
        out_shape=jax.ShapeDtypeStruct((HTOT,), jnp.float32),
    )(hist)


def _mm_first(x, w, dinv_col):
    def body(x_ref, w_ref, dv_ref, o_ref):
        o_ref[...] = dv_ref[...] * jnp.dot(
            x_ref[...], w_ref[...], preferred_element_type=jnp.float32)
    return pl.pallas_call(
        body,
        grid=(N // BR,),
        in_specs=[pl.BlockSpec((BR, D), lambda i: (i, 0)),
                  pl.BlockSpec((D, D), lambda i: (0, 0)),
                  pl.BlockSpec((BR, 1), lambda i: (i, 0))],
        out_specs=pl.BlockSpec((BR, D), lambda i: (i, 0)),
        out_shape=jax.ShapeDtypeStruct((N, D), jnp.float32),
    )(x, w, dinv_col)


def _mm_mid(s0, s1, hp, dinv_col, b_row, w):
    def body(s0_ref, s1_ref, hp_ref, dv_ref, b_ref, w_ref, o_ref):
        xl = dv_ref[...] * (s0_ref[...] + s1_ref[...] + hp_ref[...]) + b_ref[...]
        o_ref[...] = dv_ref[...] * jnp.dot(
            xl, w_ref[...], preferred_element_type=jnp.float32)
    return pl.pallas_call(
        body,
        grid=(N // BR,),
        in_specs=[pl.BlockSpec((BR, D), lambda i: (i, 0)),
                  pl.BlockSpec((BR, D), lambda i: (i, 0)),
                  pl.BlockSpec((BR, D), lambda i: (i, 0)),
                  pl.BlockSpec((BR, 1), lambda i: (i, 0)),
                  pl.BlockSpec((1, D), lambda i: (0, 0)),
                  pl.BlockSpec((D, D), lambda i: (0, 0))],
        out_specs=pl.BlockSpec((BR, D), lambda i: (i, 0)),
        out_shape=jax.ShapeDtypeStruct((N, D), jnp.float32),
    )(s0, s1, hp, dinv_col, b_row, w)


def _final(s0, s1, hp, dinv_col, b_row):
    def body(s0_ref, s1_ref, hp_ref, dv_ref, b_ref, o_ref):
        z = dv_ref[...] * (s0_ref[...] + s1_ref[...] + hp_ref[...]) + b_ref[...]
        m = jnp.max(z, axis=1, keepdims=True)
        lse = m + jnp.log(jnp.sum(jnp.exp(z - m), axis=1, keepdims=True))
        o_ref[...] = z - lse
    return pl.pallas_call(
        body,
        grid=(N // BR,),
        in_specs=[pl.BlockSpec((BR, D), lambda i: (i, 0)),
                  pl.BlockSpec((BR, D), lambda i: (i, 0)),
                  pl.BlockSpec((BR, D), lambda i: (i, 0)),
                  pl.BlockSpec((BR, 1), lambda i: (i, 0)),
                  pl.BlockSpec((1, D), lambda i: (0, 0))],
        out_specs=pl.BlockSpec((BR, D), lambda i: (i, 0)),
        out_shape=jax.ShapeDtypeStruct((N, D), jnp.float32),
    )(s0, s1, hp, dinv_col, b_row)


def kernel(x, edge_index, W0, b0, W1, b1, W2, b2):
    idx4 = jnp.stack([edge_index[0].reshape(NW, NCH, CHUNK),
                      edge_index[1].reshape(NW, NCH, CHUNK)], axis=2)
    hist = _deg_kernel()(idx4)
    dinv_col = _dinv(hist)[:N].reshape(N, 1)
    agg = _agg_kernel()
    h0 = _mm_first(x, W0, dinv_col)
    s0 = agg(h0, idx4)
    h1 = _mm_mid(s0[0, :N], s0[1, :N], h0, dinv_col, b0.reshape(1, D), W1)
    s1 = agg(h1, idx4)
    h2 = _mm_mid(s1[0, :N], s1[1, :N], h1, dinv_col, b1.reshape(1, D), W2)
    s2 = agg(h2, idx4)
    return _final(s2[0, :N], s2[1, :N], h2, dinv_col, b2.reshape(1, D))


# speedup vs baseline: 21.1324x; 1.0147x over previous
import functools

import jax
import jax.numpy as jnp
from jax import lax
from jax.experimental import pallas as pl
from jax.experimental.pallas import tpu as pltpu
from jax.experimental.pallas import tpu_sc as plsc

N = 10000
D = 128
E = 320000
NC = 2
NS = 16
NW = NC * NS
EPW = E // NW
CHUNK = 80
NCH = EPW // CHUNK
NPAD = 10240
RPS = NPAD // NS
HP = 640
HTOT = NS * HP
BR = 2000


def _mesh():
    return plsc.VectorSubcoreMesh(
        core_axis_name="c", subcore_axis_name="s",
        num_cores=NC, num_subcores=NS)


DSP = HTOT // NW


@functools.lru_cache(maxsize=None)
def _deg_kernel():
    def body(idx_hbm, out_hbm, *, idx_all, ones_v, z_v, hist):
        zero16 = jnp.broadcast_to(jnp.float32(0.0), (16,))
        ones16 = jnp.broadcast_to(jnp.float32(1.0), (16,))
        c = lax.axis_index("c")
        s = lax.axis_index("s")
        w = c * NS + s
        pltpu.sync_copy(idx_hbm.at[2 * s], idx_all.at[pl.ds(0, NCH)])
        pltpu.sync_copy(idx_hbm.at[2 * s + 1], idx_all.at[pl.ds(NCH, NCH)])
        for j in range(CHUNK // 16):
            ones_v[pl.ds(j * 16, 16)] = ones16

        def zfill(i, carry):
            z_v[pl.ds(i * 16, 16)] = zero16
            return carry
        lax.fori_loop(0, HP // 16, zfill, 0)
        pltpu.sync_copy(z_v, hist.at[pl.ds(s * HP, HP)])
        plsc.subcore_barrier()

        def step(j, carry):
            pltpu.sync_copy(ones_v, hist.at[idx_all.at[j, 1]], add=True)
            return carry
        lax.fori_loop(0, 2 * NCH, step, 0)
        plsc.subcore_barrier()
        pltpu.sync_copy(hist.at[pl.ds(w * DSP, DSP)], z_v.at[pl.ds(0, DSP)])

        def newton(i, carry):
            x = z_v[pl.ds(i * 16, 16)] + 1.0
            xi = lax.bitcast_convert_type(x, jnp.int32)
            yi = jnp.int32(0x5F3759DF) - (xi >> 1)
            y = lax.bitcast_convert_type(yi, jnp.float32)
            hx = 0.5 * x
            y = y * (1.5 - hx * y * y)
            y = y * (1.5 - hx * y * y)
            y = y * (1.5 - hx * y * y)
            z_v[pl.ds(i * 16, 16)] = y
            return carry
        lax.fori_loop(0, DSP // 16, newton, 0)
        pltpu.sync_copy(z_v.at[pl.ds(0, DSP)], out_hbm.at[pl.ds(w * DSP, DSP)])

    return pl.kernel(
        body,
        out_type=jax.ShapeDtypeStruct((HTOT,), jnp.float32),
        mesh=_mesh(),
        scratch_types=dict(
            idx_all=pltpu.VMEM((2 * NCH, 2, CHUNK), jnp.int32),
            ones_v=pltpu.VMEM((CHUNK,), jnp.float32),
            z_v=pltpu.VMEM((HP,), jnp.float32),
            hist=pltpu.VMEM_SHARED((HTOT,), jnp.float32),
        ),
    )


@functools.lru_cache(maxsize=None)
def _agg_kernel():
    nfull = RPS // CHUNK

    def body(h_hbm, idx_hbm, out_hbm, *,
             ib_a, ib_b, buf_a, buf_b, acc, sem_i, sem_a, sem_b):
        zero16 = jnp.broadcast_to(jnp.float32(0.0), (16,))
        c = lax.axis_index("c")
        s = lax.axis_index("s")
        w = c * NS + s

        def zrow(i, carry):
            for j in range(D // 16):
                buf_a[i, pl.ds(j * 16, 16)] = zero16
            return carry
        lax.fori_loop(0, CHUNK, zrow, 0)
        base = s * RPS
        for k in range(nfull):
            pltpu.sync_copy(buf_a, acc.at[pl.ds(base + k * CHUNK, CHUNK)])

        pltpu.sync_copy(idx_hbm.at[w, 0], ib_a)
        pltpu.async_copy(h_hbm.at[ib_a.at[0]], buf_a, sem_a)
        pltpu.async_copy(idx_hbm.at[w, 1], ib_b, sem_i)
        plsc.subcore_barrier()

        def step(i, carry):
            j0 = 2 * i
            pltpu.make_async_copy(idx_hbm.at[w, 0], ib_b, sem_i).wait()
            pltpu.make_async_copy(h_hbm.at[ib_a.at[0]], buf_a, sem_a).wait()
            pltpu.async_copy(h_hbm.at[ib_b.at[0]], buf_b, sem_b)
            pltpu.sync_copy(buf_a, acc.at[ib_a.at[1]], add=True)
            pltpu.async_copy(idx_hbm.at[w, j0 + 2], ib_a, sem_i)

            pltpu.make_async_copy(idx_hbm.at[w, 0], ib_a, sem_i).wait()
            pltpu.make_async_copy(h_hbm.at[ib_b.at[0]], buf_b, sem_b).wait()
            pltpu.async_copy(h_hbm.at[ib_a.at[0]], buf_a, sem_a)
            pltpu.sync_copy(buf_b, acc.at[ib_b.at[1]], add=True)
            jn = lax.min(j0 + 3, NCH - 1)
            pltpu.async_copy(idx_hbm.at[w, jn], ib_b, sem_i)
            return carry
        lax.fori_loop(0, NCH // 2, step, 0)
        pltpu.make_async_copy(idx_hbm.at[w, 0], ib_b, sem_i).wait()
        pltpu.make_async_copy(h_hbm.at[ib_a.at[0]], buf_a, sem_a).wait()
        pltpu.sync_copy(buf_a, acc.at[ib_a.at[1]], add=True)
        plsc.subcore_barrier()

        for k in range(nfull):
            off = base + k * CHUNK
            pltpu.sync_copy(acc.at[pl.ds(off, CHUNK)],
                            out_hbm.at[c, pl.ds(off, CHUNK)])

    return pl.kernel(
        body,
        out_type=jax.ShapeDtypeStruct((NC, NPAD, D), jnp.float32),
        mesh=_mesh(),
        scratch_types=dict(
            ib_a=pltpu.VMEM((2, CHUNK), jnp.int32),
            ib_b=pltpu.VMEM((2, CHUNK), jnp.int32),
            buf_a=pltpu.VMEM((CHUNK, D), jnp.float32),
            buf_b=pltpu.VMEM((CHUNK, D), jnp.float32),
            acc=pltpu.VMEM_SHARED((NPAD, D), jnp.float32),
            sem_i=pltpu.SemaphoreType.DMA,
            sem_a=pltpu.SemaphoreType.DMA,
            sem_b=pltpu.SemaphoreType.DMA,
        ),
    )


def _mm_first(x, w):
    def body(x_ref, w_ref, o_ref):
        o_ref[...] = jnp.dot(
            x_ref[...], w_ref[...], preferred_element_type=jnp.float32)
    return pl.pallas_call(
        body,
        grid=(N // BR,),
        in_specs=[pl.BlockSpec((BR, D), lambda i: (i, 0)),
                  pl.BlockSpec((D, D), lambda i: (0, 0))],
        out_specs=pl.BlockSpec((BR, D), lambda i: (i, 0)),
        out_shape=jax.ShapeDtypeStruct((N, D), jnp.float32),
    )(x, w)


def _scale(h, dinv_col):
    def body(h_ref, dv_ref, o_ref):
        o_ref[...] = dv_ref[...] * h_ref[...]
    return pl.pallas_call(
        body,
        grid=(N // BR,),
        in_specs=[pl.BlockSpec((BR, D), lambda i: (i, 0)),
                  pl.BlockSpec((BR, 1), lambda i: (i, 0))],
        out_specs=pl.BlockSpec((BR, D), lambda i: (i, 0)),
        out_shape=jax.ShapeDtypeStruct((N, D), jnp.float32),
    )(h, dinv_col)


def _mm_mid(s, hp, dinv_col, b_row, w):
    def body(s_ref, hp_ref, dv_ref, b_ref, w_ref, o_ref):
        xl = dv_ref[...] * (s_ref[0] + s_ref[1] + hp_ref[...]) + b_ref[...]
        o_ref[...] = dv_ref[...] * jnp.dot(
            xl, w_ref[...], preferred_element_type=jnp.float32)
    return pl.pallas_call(
        body,
        grid=(N // BR,),
        in_specs=[pl.BlockSpec((NC, BR, D), lambda i: (0, i, 0)),
                  pl.BlockSpec((BR, D), lambda i: (i, 0)),
                  pl.BlockSpec((BR, 1), lambda i: (i, 0)),
                  pl.BlockSpec((1, D), lambda i: (0, 0)),
                  pl.BlockSpec((D, D), lambda i: (0, 0))],
        out_specs=pl.BlockSpec((BR, D), lambda i: (i, 0)),
        out_shape=jax.ShapeDtypeStruct((N, D), jnp.float32),
    )(s, hp, dinv_col, b_row, w)


def _final(s, hp, dinv_col, b_row):
    def body(s_ref, hp_ref, dv_ref, b_ref, o_ref):
        z = dv_ref[...] * (s_ref[0] + s_ref[1] + hp_ref[...]) + b_ref[...]
        m = jnp.max(z, axis=1, keepdims=True)
        lse = m + jnp.log(jnp.sum(jnp.exp(z - m), axis=1, keepdims=True))
        o_ref[...] = z - lse
    return pl.pallas_call(
        body,
        grid=(N // BR,),
        in_specs=[pl.BlockSpec((NC, BR, D), lambda i: (0, i, 0)),
                  pl.BlockSpec((BR, D), lambda i: (i, 0)),
                  pl.BlockSpec((BR, 1), lambda i: (i, 0)),
                  pl.BlockSpec((1, D), lambda i: (0, 0))],
        out_specs=pl.BlockSpec((BR, D), lambda i: (i, 0)),
        out_shape=jax.ShapeDtypeStruct((N, D), jnp.float32),
    )(s, hp, dinv_col, b_row)


def kernel(x, edge_index, W0, b0, W1, b1, W2, b2):
    idx4 = jnp.stack([edge_index[0].reshape(NW, NCH, CHUNK),
                      edge_index[1].reshape(NW, NCH, CHUNK)], axis=2)
    dinv_flat = _deg_kernel()(idx4)
    h0_raw = _mm_first(x, W0)
    dinv_col = dinv_flat[:N].reshape(N, 1)
    agg = _agg_kernel()
    h0 = _scale(h0_raw, dinv_col)
    s0 = agg(h0, idx4)
    h1 = _mm_mid(s0, h0, dinv_col, b0.reshape(1, D), W1)
    s1 = agg(h1, idx4)
    h2 = _mm_mid(s1, h1, dinv_col, b1.reshape(1, D), W2)
    s2 = agg(h2, idx4)
    return _final(s2, h2, dinv_col, b2.reshape(1, D))


# speedup vs baseline: 29.6250x; 1.4019x over previous
import functools

import jax
import jax.numpy as jnp
from jax import lax
from jax.experimental import pallas as pl
from jax.experimental.pallas import tpu as pltpu
from jax.experimental.pallas import tpu_sc as plsc

N = 10000
D = 128
E = 320000
NC = 2
NS = 16
NW = NC * NS
EPW = E // NW
CHUNK = 80
NCH = EPW // CHUNK
NPAD = 10240
RPS = NPAD // NS
HP = 640
HTOT = NS * HP
BR = 2000


def _mesh():
    return plsc.VectorSubcoreMesh(
        core_axis_name="c", subcore_axis_name="s",
        num_cores=NC, num_subcores=NS)


DSP = HTOT // NW


@functools.lru_cache(maxsize=None)
def _deg_kernel():
    def body(idx_hbm, out_hbm, *, idx_all, ones_v, z_v, hist):
        zero16 = jnp.broadcast_to(jnp.float32(0.0), (16,))
        ones16 = jnp.broadcast_to(jnp.float32(1.0), (16,))
        c = lax.axis_index("c")
        s = lax.axis_index("s")
        w = c * NS + s
        pltpu.sync_copy(idx_hbm.at[2 * s], idx_all.at[pl.ds(0, NCH)])
        pltpu.sync_copy(idx_hbm.at[2 * s + 1], idx_all.at[pl.ds(NCH, NCH)])
        for j in range(CHUNK // 16):
            ones_v[pl.ds(j * 16, 16)] = ones16

        def zfill(i, carry):
            z_v[pl.ds(i * 16, 16)] = zero16
            return carry
        lax.fori_loop(0, HP // 16, zfill, 0)
        pltpu.sync_copy(z_v, hist.at[pl.ds(s * HP, HP)])
        plsc.subcore_barrier()

        def step(j, carry):
            pltpu.sync_copy(ones_v, hist.at[idx_all.at[j, 1]], add=True)
            return carry
        lax.fori_loop(0, 2 * NCH, step, 0)
        plsc.subcore_barrier()
        pltpu.sync_copy(hist.at[pl.ds(w * DSP, DSP)], z_v.at[pl.ds(0, DSP)])

        def newton(i, carry):
            x = z_v[pl.ds(i * 16, 16)] + 1.0
            xi = lax.bitcast_convert_type(x, jnp.int32)
            yi = jnp.int32(0x5F3759DF) - (xi >> 1)
            y = lax.bitcast_convert_type(yi, jnp.float32)
            hx = 0.5 * x
            y = y * (1.5 - hx * y * y)
            y = y * (1.5 - hx * y * y)
            y = y * (1.5 - hx * y * y)
            z_v[pl.ds(i * 16, 16)] = y
            return carry
        lax.fori_loop(0, DSP // 16, newton, 0)
        pltpu.sync_copy(z_v.at[pl.ds(0, DSP)], out_hbm.at[pl.ds(w * DSP, DSP)])

    return pl.kernel(
        body,
        out_type=jax.ShapeDtypeStruct((HTOT,), jnp.float32),
        mesh=_mesh(),
        scratch_types=dict(
            idx_all=pltpu.VMEM((2 * NCH, 2, CHUNK), jnp.int32),
            ones_v=pltpu.VMEM((CHUNK,), jnp.float32),
            z_v=pltpu.VMEM((HP,), jnp.float32),
            hist=pltpu.VMEM_SHARED((HTOT,), jnp.float32),
        ),
    )


@functools.lru_cache(maxsize=None)
def _agg_kernel():
    nfull = RPS // CHUNK

    def body(h_hbm, idx_hbm, out_hbm, *, ibs, bufs, acc, si, sg, ss):
        zero16 = jnp.broadcast_to(jnp.float32(0.0), (16,))
        c = lax.axis_index("c")
        s = lax.axis_index("s")
        w = c * NS + s

        def fire_idx(j, m):
            pltpu.async_copy(idx_hbm.at[w, j], ibs[m], si[m])

        def wait_idx(m):
            pltpu.make_async_copy(idx_hbm.at[w, 0], ibs[m], si[m]).wait()

        def fire_gather(m, k):
            pltpu.async_copy(h_hbm.at[ibs[m].at[0]], bufs[k], sg[k])

        def wait_gather(k):
            pltpu.make_async_copy(h_hbm.at[ibs[0].at[0]], bufs[k], sg[k]).wait()

        def fire_scatter(k, m):
            pltpu.async_copy(bufs[k], acc.at[ibs[m].at[1]], ss[k], add=True)

        def wait_scatter(k):
            pltpu.make_async_copy(bufs[k], acc.at[ibs[0].at[1]], ss[k]).wait()

        def zrow(i, carry):
            for j in range(D // 16):
                bufs[0][i, pl.ds(j * 16, 16)] = zero16
            return carry
        lax.fori_loop(0, CHUNK, zrow, 0)
        base = s * RPS
        for k in range(nfull):
            pltpu.sync_copy(bufs[0], acc.at[pl.ds(base + k * CHUNK, CHUNK)])

        for j in range(4):
            fire_idx(j, j)
        wait_idx(0)
        fire_gather(0, 0)
        wait_idx(1)
        fire_gather(1, 1)
        plsc.subcore_barrier()

        wait_gather(0)
        fire_scatter(0, 0)
        fire_idx(4, 4)
        wait_idx(2)
        fire_gather(2, 2)

        def step(i, carry):
            jb = 1 + 6 * i
            for u in range(6):
                k = (1 + u) % 3
                m = (1 + u) % 6
                k2 = (k + 2) % 3
                m2 = (m + 2) % 6
                m4 = (m + 4) % 6
                wait_gather(k)
                fire_scatter(k, m)
                wait_scatter(k2)
                fire_idx(jb + u + 4, m4)
                wait_idx(m2)
                fire_gather(m2, k2)
            return carry
        lax.fori_loop(0, 20, step, 0)

        wait_gather(1)
        fire_scatter(1, 1)
        wait_scatter(0)
        wait_idx(3)
        fire_gather(3, 0)

        wait_gather(2)
        fire_scatter(2, 2)
        wait_scatter(1)
        wait_idx(4)
        fire_gather(4, 1)

        wait_gather(0)
        fire_scatter(0, 3)
        wait_gather(1)
        fire_scatter(1, 4)
        wait_scatter(2)
        wait_scatter(0)
        wait_scatter(1)
        plsc.subcore_barrier()

        for k in range(nfull):
            off = base + k * CHUNK
            pltpu.sync_copy(acc.at[pl.ds(off, CHUNK)],
                            out_hbm.at[c, pl.ds(off, CHUNK)])

    return pl.kernel(
        body,
        out_type=jax.ShapeDtypeStruct((NC, NPAD, D), jnp.float32),
        mesh=_mesh(),
        scratch_types=dict(
            ibs=tuple(pltpu.VMEM((2, CHUNK), jnp.int32) for _ in range(6)),
            bufs=tuple(pltpu.VMEM((CHUNK, D), jnp.float32) for _ in range(3)),
            acc=pltpu.VMEM_SHARED((NPAD, D), jnp.float32),
            si=tuple(pltpu.SemaphoreType.DMA for _ in range(6)),
            sg=tuple(pltpu.SemaphoreType.DMA for _ in range(3)),
            ss=tuple(pltpu.SemaphoreType.DMA for _ in range(3)),
        ),
    )


def _mm_first(x, w):
    def body(x_ref, w_ref, o_ref):
        o_ref[...] = jnp.dot(
            x_ref[...], w_ref[...], preferred_element_type=jnp.float32)
    return pl.pallas_call(
        body,
        grid=(N // BR,),
        in_specs=[pl.BlockSpec((BR, D), lambda i: (i, 0)),
                  pl.BlockSpec((D, D), lambda i: (0, 0))],
        out_specs=pl.BlockSpec((BR, D), lambda i: (i, 0)),
        out_shape=jax.ShapeDtypeStruct((N, D), jnp.float32),
    )(x, w)


def _scale(h, dinv_col):
    def body(h_ref, dv_ref, o_ref):
        o_ref[...] = dv_ref[...] * h_ref[...]
    return pl.pallas_call(
        body,
        grid=(N // BR,),
        in_specs=[pl.BlockSpec((BR, D), lambda i: (i, 0)),
                  pl.BlockSpec((BR, 1), lambda i: (i, 0))],
        out_specs=pl.BlockSpec((BR, D), lambda i: (i, 0)),
        out_shape=jax.ShapeDtypeStruct((N, D), jnp.float32),
    )(h, dinv_col)


def _mm_mid(s, hp, dinv_col, b_row, w):
    def body(s_ref, hp_ref, dv_ref, b_ref, w_ref, o_ref):
        xl = dv_ref[...] * (s_ref[0] + s_ref[1] + hp_ref[...]) + b_ref[...]
        o_ref[...] = dv_ref[...] * jnp.dot(
            xl, w_ref[...], preferred_element_type=jnp.float32)
    return pl.pallas_call(
        body,
        grid=(N // BR,),
        in_specs=[pl.BlockSpec((NC, BR, D), lambda i: (0, i, 0)),
                  pl.BlockSpec((BR, D), lambda i: (i, 0)),
                  pl.BlockSpec((BR, 1), lambda i: (i, 0)),
                  pl.BlockSpec((1, D), lambda i: (0, 0)),
                  pl.BlockSpec((D, D), lambda i: (0, 0))],
        out_specs=pl.BlockSpec((BR, D), lambda i: (i, 0)),
        out_shape=jax.ShapeDtypeStruct((N, D), jnp.float32),
    )(s, hp, dinv_col, b_row, w)


def _final(s, hp, dinv_col, b_row):
    def body(s_ref, hp_ref, dv_ref, b_ref, o_ref):
        z = dv_ref[...] * (s_ref[0] + s_ref[1] + hp_ref[...]) + b_ref[...]
        m = jnp.max(z, axis=1, keepdims=True)
        lse = m + jnp.log(jnp.sum(jnp.exp(z - m), axis=1, keepdims=True))
        o_ref[...] = z - lse
    return pl.pallas_call(
        body,
        grid=(N // BR,),
        in_specs=[pl.BlockSpec((NC, BR, D), lambda i: (0, i, 0)),
                  pl.BlockSpec((BR, D), lambda i: (i, 0)),
                  pl.BlockSpec((BR, 1), lambda i: (i, 0)),
                  pl.BlockSpec((1, D), lambda i: (0, 0))],
        out_specs=pl.BlockSpec((BR, D), lambda i: (i, 0)),
        out_shape=jax.ShapeDtypeStruct((N, D), jnp.float32),
    )(s, hp, dinv_col, b_row)


def kernel(x, edge_index, W0, b0, W1, b1, W2, b2):
    idx4 = jnp.stack([edge_index[0].reshape(NW, NCH, CHUNK),
                      edge_index[1].reshape(NW, NCH, CHUNK)], axis=2)
    dinv_flat = _deg_kernel()(idx4)
    h0_raw = _mm_first(x, W0)
    dinv_col = dinv_flat[:N].reshape(N, 1)
    agg = _agg_kernel()
    h0 = _scale(h0_raw, dinv_col)
    s0 = agg(h0, idx4)
    h1 = _mm_mid(s0, h0, dinv_col, b0.reshape(1, D), W1)
    s1 = agg(h1, idx4)
    h2 = _mm_mid(s1, h1, dinv_col, b1.reshape(1, D), W2)
    s2 = agg(h2, idx4)
    return _final(s2, h2, dinv_col, b2.reshape(1, D))


# speedup vs baseline: 30.9889x; 1.0460x over previous
import functools

import jax
import jax.numpy as jnp
from jax import lax
from jax.experimental import pallas as pl
from jax.experimental.pallas import tpu as pltpu
from jax.experimental.pallas import tpu_sc as plsc

N = 10000
D = 128
E = 320000
NC = 2
NS = 16
NW = NC * NS
EPW = E // NW
CHUNK = 80
NCH = EPW // CHUNK
NPAD = 10240
RPS = NPAD // NS
HP = 640
HTOT = NS * HP
BR = 2000


def _mesh():
    return plsc.VectorSubcoreMesh(
        core_axis_name="c", subcore_axis_name="s",
        num_cores=NC, num_subcores=NS)


DSP = HTOT // NW


@functools.lru_cache(maxsize=None)
def _deg_kernel():
    def body(idx_hbm, out_hbm, *, idx_all, ones_v, z_v, hist, sse):
        zero16 = jnp.broadcast_to(jnp.float32(0.0), (16,))
        ones16 = jnp.broadcast_to(jnp.float32(1.0), (16,))
        c = lax.axis_index("c")
        s = lax.axis_index("s")
        w = c * NS + s
        pltpu.sync_copy(idx_hbm.at[2 * s], idx_all.at[pl.ds(0, NCH)])
        pltpu.sync_copy(idx_hbm.at[2 * s + 1], idx_all.at[pl.ds(NCH, NCH)])
        for j in range(CHUNK // 16):
            ones_v[pl.ds(j * 16, 16)] = ones16

        def zfill(i, carry):
            z_v[pl.ds(i * 16, 16)] = zero16
            return carry
        lax.fori_loop(0, HP // 16, zfill, 0)
        pltpu.sync_copy(z_v, hist.at[pl.ds(s * HP, HP)])
        plsc.subcore_barrier()

        def fire(j):
            pltpu.async_copy(ones_v, hist.at[idx_all.at[j, 1]], sse, add=True)

        def drain():
            pltpu.make_async_copy(ones_v, hist.at[idx_all.at[0, 1]], sse).wait()

        for j in range(8):
            fire(j)

        def step(j, carry):
            fire(j)
            drain()
            return carry
        lax.fori_loop(8, 2 * NCH, step, 0)
        for _ in range(8):
            drain()
        plsc.subcore_barrier()
        pltpu.sync_copy(hist.at[pl.ds(w * DSP, DSP)], z_v.at[pl.ds(0, DSP)])

        def newton(i, carry):
            x = z_v[pl.ds(i * 16, 16)] + 1.0
            xi = lax.bitcast_convert_type(x, jnp.int32)
            yi = jnp.int32(0x5F3759DF) - (xi >> 1)
            y = lax.bitcast_convert_type(yi, jnp.float32)
            hx = 0.5 * x
            y = y * (1.5 - hx * y * y)
            y = y * (1.5 - hx * y * y)
            y = y * (1.5 - hx * y * y)
            z_v[pl.ds(i * 16, 16)] = y
            return carry
        lax.fori_loop(0, DSP // 16, newton, 0)
        pltpu.sync_copy(z_v.at[pl.ds(0, DSP)], out_hbm.at[pl.ds(w * DSP, DSP)])

    return pl.kernel(
        body,
        out_type=jax.ShapeDtypeStruct((HTOT,), jnp.float32),
        mesh=_mesh(),
        scratch_types=dict(
            idx_all=pltpu.VMEM((2 * NCH, 2, CHUNK), jnp.int32),
            ones_v=pltpu.VMEM((CHUNK,), jnp.float32),
            z_v=pltpu.VMEM((HP,), jnp.float32),
            hist=pltpu.VMEM_SHARED((HTOT,), jnp.float32),
            sse=pltpu.SemaphoreType.DMA,
        ),
    )


@functools.lru_cache(maxsize=None)
def _agg_kernel():
    nfull = RPS // CHUNK

    def body(h_hbm, idx_hbm, out_hbm, *, ibs, bufs, acc, si, sg, ss):
        zero16 = jnp.broadcast_to(jnp.float32(0.0), (16,))
        c = lax.axis_index("c")
        s = lax.axis_index("s")
        w = c * NS + s

        def fire_idx(j, m):
            pltpu.async_copy(idx_hbm.at[w, j], ibs[m], si[m])

        def wait_idx(m):
            pltpu.make_async_copy(idx_hbm.at[w, 0], ibs[m], si[m]).wait()

        def fire_gather(m, k):
            pltpu.async_copy(h_hbm.at[ibs[m].at[0]], bufs[k], sg[k])

        def wait_gather(k):
            pltpu.make_async_copy(h_hbm.at[ibs[0].at[0]], bufs[k], sg[k]).wait()

        def fire_scatter(k, m):
            pltpu.async_copy(bufs[k], acc.at[ibs[m].at[1]], ss[k], add=True)

        def wait_scatter(k):
            pltpu.make_async_copy(bufs[k], acc.at[ibs[0].at[1]], ss[k]).wait()

        for j in range(4):
            fire_idx(j, j)

        def zrow(i, carry):
            for j in range(D // 16):
                bufs[0][i, pl.ds(j * 16, 16)] = zero16
            return carry
        lax.fori_loop(0, CHUNK, zrow, 0)
        base = s * RPS
        for k in range(nfull):
            pltpu.sync_copy(bufs[0], acc.at[pl.ds(base + k * CHUNK, CHUNK)])

        wait_idx(0)
        fire_gather(0, 0)
        wait_idx(1)
        fire_gather(1, 1)
        plsc.subcore_barrier()

        wait_gather(0)
        fire_scatter(0, 0)
        fire_idx(4, 4)
        wait_idx(2)
        fire_gather(2, 2)

        def step(i, carry):
            jb = 1 + 6 * i
            for u in range(6):
                k = (1 + u) % 3
                m = (1 + u) % 6
                k2 = (k + 2) % 3
                m2 = (m + 2) % 6
                m4 = (m + 4) % 6
                wait_gather(k)
                fire_scatter(k, m)
                wait_scatter(k2)
                fire_idx(jb + u + 4, m4)
                wait_idx(m2)
                fire_gather(m2, k2)
            return carry
        lax.fori_loop(0, 20, step, 0)

        wait_gather(1)
        fire_scatter(1, 1)
        wait_scatter(0)
        wait_idx(3)
        fire_gather(3, 0)

        wait_gather(2)
        fire_scatter(2, 2)
        wait_scatter(1)
        wait_idx(4)
        fire_gather(4, 1)

        wait_gather(0)
        fire_scatter(0, 3)
        wait_gather(1)
        fire_scatter(1, 4)
        wait_scatter(2)
        wait_scatter(0)
        wait_scatter(1)
        plsc.subcore_barrier()

        for k in range(nfull):
            off = base + k * CHUNK
            pltpu.sync_copy(acc.at[pl.ds(off, CHUNK)],
                            out_hbm.at[c, pl.ds(off, CHUNK)])

    return pl.kernel(
        body,
        out_type=jax.ShapeDtypeStruct((NC, NPAD, D), jnp.float32),
        mesh=_mesh(),
        scratch_types=dict(
            ibs=tuple(pltpu.VMEM((2, CHUNK), jnp.int32) for _ in range(6)),
            bufs=tuple(pltpu.VMEM((CHUNK, D), jnp.float32) for _ in range(3)),
            acc=pltpu.VMEM_SHARED((NPAD, D), jnp.float32),
            si=tuple(pltpu.SemaphoreType.DMA for _ in range(6)),
            sg=tuple(pltpu.SemaphoreType.DMA for _ in range(3)),
            ss=tuple(pltpu.SemaphoreType.DMA for _ in range(3)),
        ),
    )


def _mm_first(x, w):
    def body(x_ref, w_ref, o_ref):
        o_ref[...] = jnp.dot(
            x_ref[...], w_ref[...], preferred_element_type=jnp.float32)
    return pl.pallas_call(
        body,
        grid=(N // BR,),
        in_specs=[pl.BlockSpec((BR, D), lambda i: (i, 0)),
                  pl.BlockSpec((D, D), lambda i: (0, 0))],
        out_specs=pl.BlockSpec((BR, D), lambda i: (i, 0)),
        out_shape=jax.ShapeDtypeStruct((N, D), jnp.float32),
    )(x, w)


def _scale(h, dinv_col):
    def body(h_ref, dv_ref, o_ref):
        o_ref[...] = dv_ref[...] * h_ref[...]
    return pl.pallas_call(
        body,
        grid=(N // BR,),
        in_specs=[pl.BlockSpec((BR, D), lambda i: (i, 0)),
                  pl.BlockSpec((BR, 1), lambda i: (i, 0))],
        out_specs=pl.BlockSpec((BR, D), lambda i: (i, 0)),
        out_shape=jax.ShapeDtypeStruct((N, D), jnp.float32),
    )(h, dinv_col)


def _mm_mid(s, hp, dinv_col, b_row, w):
    def body(s_ref, hp_ref, dv_ref, b_ref, w_ref, o_ref):
        xl = dv_ref[...] * (s_ref[0] + s_ref[1] + hp_ref[...]) + b_ref[...]
        o_ref[...] = dv_ref[...] * jnp.dot(
            xl, w_ref[...], preferred_element_type=jnp.float32)
    return pl.pallas_call(
        body,
        grid=(N // BR,),
        in_specs=[pl.BlockSpec((NC, BR, D), lambda i: (0, i, 0)),
                  pl.BlockSpec((BR, D), lambda i: (i, 0)),
                  pl.BlockSpec((BR, 1), lambda i: (i, 0)),
                  pl.BlockSpec((1, D), lambda i: (0, 0)),
                  pl.BlockSpec((D, D), lambda i: (0, 0))],
        out_specs=pl.BlockSpec((BR, D), lambda i: (i, 0)),
        out_shape=jax.ShapeDtypeStruct((N, D), jnp.float32),
    )(s, hp, dinv_col, b_row, w)


def _final(s, hp, dinv_col, b_row):
    def body(s_ref, hp_ref, dv_ref, b_ref, o_ref):
        z = dv_ref[...] * (s_ref[0] + s_ref[1] + hp_ref[...]) + b_ref[...]
        m = jnp.max(z, axis=1, keepdims=True)
        lse = m + jnp.log(jnp.sum(jnp.exp(z - m), axis=1, keepdims=True))
        o_ref[...] = z - lse
    return pl.pallas_call(
        body,
        grid=(N // BR,),
        in_specs=[pl.BlockSpec((NC, BR, D), lambda i: (0, i, 0)),
                  pl.BlockSpec((BR, D), lambda i: (i, 0)),
                  pl.BlockSpec((BR, 1), lambda i: (i, 0)),
                  pl.BlockSpec((1, D), lambda i: (0, 0))],
        out_specs=pl.BlockSpec((BR, D), lambda i: (i, 0)),
        out_shape=jax.ShapeDtypeStruct((N, D), jnp.float32),
    )(s, hp, dinv_col, b_row)


def kernel(x, edge_index, W0, b0, W1, b1, W2, b2):
    idx4 = jnp.stack([edge_index[0].reshape(NW, NCH, CHUNK),
                      edge_index[1].reshape(NW, NCH, CHUNK)], axis=2)
    dinv_flat = _deg_kernel()(idx4)
    h0_raw = _mm_first(x, W0)
    dinv_col = dinv_flat[:N].reshape(N, 1)
    agg = _agg_kernel()
    h0 = _scale(h0_raw, dinv_col)
    s0 = agg(h0, idx4)
    h1 = _mm_mid(s0, h0, dinv_col, b0.reshape(1, D), W1)
    s1 = agg(h1, idx4)
    h2 = _mm_mid(s1, h1, dinv_col, b1.reshape(1, D), W2)
    s2 = agg(h2, idx4)
    return _final(s2, h2, dinv_col, b2.reshape(1, D))


# speedup vs baseline: 33.0077x; 1.0651x over previous
import functools

import jax
import jax.numpy as jnp
from jax import lax
from jax.experimental import pallas as pl
from jax.experimental.pallas import tpu as pltpu
from jax.experimental.pallas import tpu_sc as plsc

N = 10000
D = 128
E = 320000
NC = 2
NS = 16
NW = NC * NS
EPW = E // NW
CHUNK = 80
NCH = EPW // CHUNK
NPAD = 10240
RPS = NPAD // NS
HP = 640
HTOT = NS * HP
BR = 2000


def _mesh():
    return plsc.VectorSubcoreMesh(
        core_axis_name="c", subcore_axis_name="s",
        num_cores=NC, num_subcores=NS)


DSP = HTOT // NW


@functools.lru_cache(maxsize=None)
def _deg_kernel():
    def body(idx_hbm, out_hbm, *, idx_all, ones_v, z_v, hist, sse):
        zero16 = jnp.broadcast_to(jnp.float32(0.0), (16,))
        ones16 = jnp.broadcast_to(jnp.float32(1.0), (16,))
        c = lax.axis_index("c")
        s = lax.axis_index("s")
        w = c * NS + s
        pltpu.sync_copy(idx_hbm.at[1, 2 * s], idx_all.at[pl.ds(0, NCH)])
        pltpu.sync_copy(idx_hbm.at[1, 2 * s + 1], idx_all.at[pl.ds(NCH, NCH)])
        for j in range(CHUNK // 16):
            ones_v[pl.ds(j * 16, 16)] = ones16

        def zfill(i, carry):
            z_v[pl.ds(i * 16, 16)] = zero16
            return carry
        lax.fori_loop(0, HP // 16, zfill, 0)
        pltpu.sync_copy(z_v, hist.at[pl.ds(s * HP, HP)])
        plsc.subcore_barrier()

        def fire(j):
            pltpu.async_copy(ones_v, hist.at[idx_all.at[j]], sse, add=True)

        def drain():
            pltpu.make_async_copy(ones_v, hist.at[idx_all.at[0]], sse).wait()

        for j in range(8):
            fire(j)

        def step(j, carry):
            fire(j)
            drain()
            return carry
        lax.fori_loop(8, 2 * NCH, step, 0)
        for _ in range(8):
            drain()
        plsc.subcore_barrier()
        pltpu.sync_copy(hist.at[pl.ds(w * DSP, DSP)], z_v.at[pl.ds(0, DSP)])

        def newton(i, carry):
            x = z_v[pl.ds(i * 16, 16)] + 1.0
            xi = lax.bitcast_convert_type(x, jnp.int32)
            yi = jnp.int32(0x5F3759DF) - (xi >> 1)
            y = lax.bitcast_convert_type(yi, jnp.float32)
            hx = 0.5 * x
            y = y * (1.5 - hx * y * y)
            y = y * (1.5 - hx * y * y)
            y = y * (1.5 - hx * y * y)
            z_v[pl.ds(i * 16, 16)] = y
            return carry
        lax.fori_loop(0, DSP // 16, newton, 0)
        pltpu.sync_copy(z_v.at[pl.ds(0, DSP)], out_hbm.at[pl.ds(w * DSP, DSP)])

    return pl.kernel(
        body,
        out_type=jax.ShapeDtypeStruct((HTOT,), jnp.float32),
        mesh=_mesh(),
        scratch_types=dict(
            idx_all=pltpu.VMEM((2 * NCH, CHUNK), jnp.int32),
            ones_v=pltpu.VMEM((CHUNK,), jnp.float32),
            z_v=pltpu.VMEM((HP,), jnp.float32),
            hist=pltpu.VMEM_SHARED((HTOT,), jnp.float32),
            sse=pltpu.SemaphoreType.DMA,
        ),
    )


@functools.lru_cache(maxsize=None)
def _agg_kernel():
    nfull = RPS // CHUNK

    def body(h_hbm, idx_hbm, out_hbm, *, ibs, bufs, acc, si, sg, ss):
        zero16 = jnp.broadcast_to(jnp.float32(0.0), (16,))
        c = lax.axis_index("c")
        s = lax.axis_index("s")
        w = c * NS + s

        def fire_idx(j, m):
            pltpu.async_copy(idx_hbm.at[0, w, j], ibs[m].at[0], si[m])
            pltpu.async_copy(idx_hbm.at[1, w, j], ibs[m].at[1], si[m])

        def wait_idx(m):
            pltpu.make_async_copy(idx_hbm.at[0, w, 0], ibs[m].at[0], si[m]).wait()
            pltpu.make_async_copy(idx_hbm.at[0, w, 0], ibs[m].at[1], si[m]).wait()

        def fire_gather(m, k):
            pltpu.async_copy(h_hbm.at[ibs[m].at[0]], bufs[k], sg[k])

        def wait_gather(k):
            pltpu.make_async_copy(h_hbm.at[ibs[0].at[0]], bufs[k], sg[k]).wait()

        def fire_scatter(k, m):
            pltpu.async_copy(bufs[k], acc.at[ibs[m].at[1]], ss[k], add=True)

        def wait_scatter(k):
            pltpu.make_async_copy(bufs[k], acc.at[ibs[0].at[1]], ss[k]).wait()

        for j in range(4):
            fire_idx(j, j)

        def zrow(i, carry):
            for j in range(D // 16):
                bufs[0][i, pl.ds(j * 16, 16)] = zero16
            return carry
        lax.fori_loop(0, CHUNK, zrow, 0)
        base = s * RPS
        for k in range(nfull):
            pltpu.sync_copy(bufs[0], acc.at[pl.ds(base + k * CHUNK, CHUNK)])

        wait_idx(0)
        fire_gather(0, 0)
        wait_idx(1)
        fire_gather(1, 1)
        plsc.subcore_barrier()

        wait_gather(0)
        fire_scatter(0, 0)
        fire_idx(4, 4)
        wait_idx(2)
        fire_gather(2, 2)

        def step(i, carry):
            jb = 1 + 6 * i
            for u in range(6):
                k = (1 + u) % 3
                m = (1 + u) % 6
                k2 = (k + 2) % 3
                m2 = (m + 2) % 6
                m4 = (m + 4) % 6
                wait_gather(k)
                fire_scatter(k, m)
                wait_scatter(k2)
                fire_idx(jb + u + 4, m4)
                wait_idx(m2)
                fire_gather(m2, k2)
            return carry
        lax.fori_loop(0, 20, step, 0)

        wait_gather(1)
        fire_scatter(1, 1)
        wait_scatter(0)
        wait_idx(3)
        fire_gather(3, 0)

        wait_gather(2)
        fire_scatter(2, 2)
        wait_scatter(1)
        wait_idx(4)
        fire_gather(4, 1)

        wait_gather(0)
        fire_scatter(0, 3)
        wait_gather(1)
        fire_scatter(1, 4)
        wait_scatter(2)
        wait_scatter(0)
        wait_scatter(1)
        plsc.subcore_barrier()

        for k in range(nfull):
            off = base + k * CHUNK
            pltpu.sync_copy(acc.at[pl.ds(off, CHUNK)],
                            out_hbm.at[c, pl.ds(off, CHUNK)])

    return pl.kernel(
        body,
        out_type=jax.ShapeDtypeStruct((NC, NPAD, D), jnp.float32),
        mesh=_mesh(),
        scratch_types=dict(
            ibs=tuple(pltpu.VMEM((2, CHUNK), jnp.int32) for _ in range(6)),
            bufs=tuple(pltpu.VMEM((CHUNK, D), jnp.float32) for _ in range(3)),
            acc=pltpu.VMEM_SHARED((NPAD, D), jnp.float32),
            si=tuple(pltpu.SemaphoreType.DMA for _ in range(6)),
            sg=tuple(pltpu.SemaphoreType.DMA for _ in range(3)),
            ss=tuple(pltpu.SemaphoreType.DMA for _ in range(3)),
        ),
    )


def _mm_first(x, w):
    def body(x_ref, w_ref, o_ref):
        o_ref[...] = jnp.dot(
            x_ref[...], w_ref[...], preferred_element_type=jnp.float32)
    return pl.pallas_call(
        body,
        grid=(N // BR,),
        in_specs=[pl.BlockSpec((BR, D), lambda i: (i, 0)),
                  pl.BlockSpec((D, D), lambda i: (0, 0))],
        out_specs=pl.BlockSpec((BR, D), lambda i: (i, 0)),
        out_shape=jax.ShapeDtypeStruct((N, D), jnp.float32),
    )(x, w)


def _scale(h, dinv_col):
    def body(h_ref, dv_ref, o_ref):
        o_ref[...] = dv_ref[...] * h_ref[...]
    return pl.pallas_call(
        body,
        grid=(N // BR,),
        in_specs=[pl.BlockSpec((BR, D), lambda i: (i, 0)),
                  pl.BlockSpec((BR, 1), lambda i: (i, 0))],
        out_specs=pl.BlockSpec((BR, D), lambda i: (i, 0)),
        out_shape=jax.ShapeDtypeStruct((N, D), jnp.float32),
    )(h, dinv_col)


def _mm_mid(s, hp, dinv_col, b_row, w):
    def body(s_ref, hp_ref, dv_ref, b_ref, w_ref, o_ref):
        xl = dv_ref[...] * (s_ref[0] + s_ref[1] + hp_ref[...]) + b_ref[...]
        o_ref[...] = dv_ref[...] * jnp.dot(
            xl, w_ref[...], preferred_element_type=jnp.float32)
    return pl.pallas_call(
        body,
        grid=(N // BR,),
        in_specs=[pl.BlockSpec((NC, BR, D), lambda i: (0, i, 0)),
                  pl.BlockSpec((BR, D), lambda i: (i, 0)),
                  pl.BlockSpec((BR, 1), lambda i: (i, 0)),
                  pl.BlockSpec((1, D), lambda i: (0, 0)),
                  pl.BlockSpec((D, D), lambda i: (0, 0))],
        out_specs=pl.BlockSpec((BR, D), lambda i: (i, 0)),
        out_shape=jax.ShapeDtypeStruct((N, D), jnp.float32),
    )(s, hp, dinv_col, b_row, w)


def _final(s, hp, dinv_col, b_row):
    def body(s_ref, hp_ref, dv_ref, b_ref, o_ref):
        z = dv_ref[...] * (s_ref[0] + s_ref[1] + hp_ref[...]) + b_ref[...]
        m = jnp.max(z, axis=1, keepdims=True)
        lse = m + jnp.log(jnp.sum(jnp.exp(z - m), axis=1, keepdims=True))
        o_ref[...] = z - lse
    return pl.pallas_call(
        body,
        grid=(N // BR,),
        in_specs=[pl.BlockSpec((NC, BR, D), lambda i: (0, i, 0)),
                  pl.BlockSpec((BR, D), lambda i: (i, 0)),
                  pl.BlockSpec((BR, 1), lambda i: (i, 0)),
                  pl.BlockSpec((1, D), lambda i: (0, 0))],
        out_specs=pl.BlockSpec((BR, D), lambda i: (i, 0)),
        out_shape=jax.ShapeDtypeStruct((N, D), jnp.float32),
    )(s, hp, dinv_col, b_row)


def kernel(x, edge_index, W0, b0, W1, b1, W2, b2):
    idx4 = edge_index.reshape(2, NW, NCH, CHUNK)
    dinv_flat = _deg_kernel()(idx4)
    h0_raw = _mm_first(x, W0)
    dinv_col = dinv_flat[:N].reshape(N, 1)
    agg = _agg_kernel()
    h0 = _scale(h0_raw, dinv_col)
    s0 = agg(h0, idx4)
    h1 = _mm_mid(s0, h0, dinv_col, b0.reshape(1, D), W1)
    s1 = agg(h1, idx4)
    h2 = _mm_mid(s1, h1, dinv_col, b1.reshape(1, D), W2)
    s2 = agg(h2, idx4)
    return _final(s2, h2, dinv_col, b2.reshape(1, D))
